# baseline (speedup 1.0000x reference)
import functools

import jax
import jax.numpy as jnp
from jax import lax
from jax.experimental import pallas as pl
from jax.experimental.pallas import tpu as pltpu
from jax.experimental.pallas import tpu_sc as plsc

N_NODES = 10000
N_EDGES = 320000
D_FEAT = 128
N_CLASSES = 16

NC = 2
NS = 16
NW = NC * NS
CHUNK = 128
K = 80
KG = 8
E_PAD = NW * K * CHUNK
W_AGG = 2 * N_CLASSES
N_SP = 10112
ROWS_PER_TILE = N_SP // NS


def _proj_body(x_ref, w2_ref, out_ref):
    y = jnp.dot(x_ref[...], w2_ref[...], preferred_element_type=jnp.float32)
    col = lax.broadcasted_iota(jnp.int32, y.shape, 1)
    out_ref[...] = y + jnp.where(col == N_CLASSES, 1.0, 0.0)


def _fin_body(p0_ref, p1_ref, x_ref, wr_ref, b_ref, out_ref):
    ssum = p0_ref[...] + p1_ref[...]
    agg = ssum[:, :N_CLASSES]
    cnt = ssum[:, N_CLASSES:N_CLASSES + 1]
    mean = agg / jnp.maximum(cnt, 1.0)
    z = mean + b_ref[...] + jnp.dot(x_ref[...], wr_ref[...],
                                    preferred_element_type=jnp.float32)
    z = jnp.maximum(z, 0.0)
    m = jnp.max(z, axis=1, keepdims=True)
    lse = m + jnp.log(jnp.sum(jnp.exp(z - m), axis=1, keepdims=True))
    out_ref[...] = z - lse


def _make_sc_kernel():
    mesh = plsc.VectorSubcoreMesh(core_axis_name="c", subcore_axis_name="s",
                                  num_cores=NC, num_subcores=NS)

    @functools.partial(
        pl.kernel,
        out_type=jax.ShapeDtypeStruct((NC, N_SP, W_AGG), jnp.float32),
        mesh=mesh,
        scratch_types=[
            pltpu.VMEM((K, CHUNK), jnp.int32),
            pltpu.VMEM((K, CHUNK), jnp.int32),
            pltpu.VMEM((KG, CHUNK, W_AGG), jnp.float32),
            pltpu.VMEM((ROWS_PER_TILE, W_AGG), jnp.float32),
            pltpu.VMEM_SHARED((N_SP, W_AGG), jnp.float32),
            pltpu.SemaphoreType.DMA,
        ],
        compiler_params=pltpu.CompilerParams(use_tc_tiling_on_sc=False),
    )
    def sc_aggregate(src_hbm, dst_hbm, y2_hbm, zeros_hbm, out_hbm,
                     src_v, dst_v, rows_v, slab_v, agg_sh, sem):
        c = lax.axis_index("c")
        s = lax.axis_index("s")
        wid = s * NC + c
        row0 = s * ROWS_PER_TILE

        pltpu.sync_copy(zeros_hbm.at[pl.ds(row0, ROWS_PER_TILE)], slab_v)
        pltpu.sync_copy(slab_v, agg_sh.at[pl.ds(row0, ROWS_PER_TILE)])
        plsc.subcore_barrier()

        pltpu.sync_copy(src_hbm.at[wid], src_v)
        pltpu.sync_copy(dst_hbm.at[wid], dst_v)

        @pl.loop(0, K // KG)
        def group(g):
            base = g * KG
            descs = [
                pltpu.async_copy(y2_hbm.at[src_v.at[base + b]], rows_v.at[b], sem)
                for b in range(KG)
            ]
            for d in descs:
                d.wait()
            for b in range(KG):
                pltpu.sync_copy(rows_v.at[b], agg_sh.at[dst_v.at[base + b]],
                                add=True)
        plsc.subcore_barrier()

        pltpu.sync_copy(agg_sh.at[pl.ds(row0, ROWS_PER_TILE)], slab_v)
        pltpu.sync_copy(slab_v, out_hbm.at[c, pl.ds(row0, ROWS_PER_TILE)])

    return sc_aggregate


_SC_AGGREGATE = _make_sc_kernel()


def kernel(x, edge_index, W_l, b_l, W_r):
    ei = edge_index.astype(jnp.int32)
    pad = E_PAD - N_EDGES
    src = jnp.concatenate([ei[0], jnp.zeros((pad,), jnp.int32)])
    dst = jnp.concatenate([ei[1], jnp.full((pad,), N_NODES, jnp.int32)])
    src3 = src.reshape(NW, K, CHUNK)
    dst3 = dst.reshape(NW, K, CHUNK)

    w2 = jnp.concatenate(
        [W_l.T, jnp.zeros((D_FEAT, W_AGG - N_CLASSES), jnp.float32)], axis=1)

    blk = 400
    y2 = pl.pallas_call(
        _proj_body,
        grid=(N_NODES // blk,),
        in_specs=[
            pl.BlockSpec((blk, D_FEAT), lambda i: (i, 0)),
            pl.BlockSpec((D_FEAT, W_AGG), lambda i: (0, 0)),
        ],
        out_specs=pl.BlockSpec((blk, W_AGG), lambda i: (i, 0)),
        out_shape=jax.ShapeDtypeStruct((N_NODES, W_AGG), jnp.float32),
    )(x, w2)

    zeros = jnp.zeros((N_SP, W_AGG), jnp.float32)
    parts = _SC_AGGREGATE(src3, dst3, y2, zeros)

    p0 = parts[0, :N_NODES, :]
    p1 = parts[1, :N_NODES, :]
    out = pl.pallas_call(
        _fin_body,
        grid=(N_NODES // blk,),
        in_specs=[
            pl.BlockSpec((blk, W_AGG), lambda i: (i, 0)),
            pl.BlockSpec((blk, W_AGG), lambda i: (i, 0)),
            pl.BlockSpec((blk, D_FEAT), lambda i: (i, 0)),
            pl.BlockSpec((D_FEAT, N_CLASSES), lambda i: (0, 0)),
            pl.BlockSpec((1, N_CLASSES), lambda i: (0, 0)),
        ],
        out_specs=pl.BlockSpec((blk, N_CLASSES), lambda i: (i, 0)),
        out_shape=jax.ShapeDtypeStruct((N_NODES, N_CLASSES), jnp.float32),
    )(p0, p1, x, W_r.T, b_l.reshape(1, N_CLASSES))
    return out


# speedup vs baseline: 1.0177x; 1.0177x over previous
import functools

import jax
import jax.numpy as jnp
from jax import lax
from jax.experimental import pallas as pl
from jax.experimental.pallas import tpu as pltpu
from jax.experimental.pallas import tpu_sc as plsc

N_NODES = 10000
N_EDGES = 320000
D_FEAT = 128
N_CLASSES = 16

NC = 2
NS = 16
NW = NC * NS
CHUNK = 128
K = 80
KG = 8
E_PAD = NW * K * CHUNK
W_AGG = 2 * N_CLASSES
N_SP = 10112
ROWS_PER_TILE = N_SP // NS


def _proj_body(x_ref, w2_ref, out_ref):
    y = jnp.dot(x_ref[...], w2_ref[...], preferred_element_type=jnp.float32)
    col = lax.broadcasted_iota(jnp.int32, y.shape, 1)
    out_ref[...] = y + jnp.where(col == N_CLASSES, 1.0, 0.0)


def _fin_body(p0_ref, p1_ref, x_ref, wr_ref, b_ref, out_ref):
    ssum = p0_ref[...] + p1_ref[...]
    agg = ssum[:, :N_CLASSES]
    cnt = ssum[:, N_CLASSES:N_CLASSES + 1]
    mean = agg / jnp.maximum(cnt, 1.0)
    z = mean + b_ref[...] + jnp.dot(x_ref[...], wr_ref[...],
                                    preferred_element_type=jnp.float32)
    z = jnp.maximum(z, 0.0)
    m = jnp.max(z, axis=1, keepdims=True)
    lse = m + jnp.log(jnp.sum(jnp.exp(z - m), axis=1, keepdims=True))
    out_ref[...] = z - lse


def _make_sc_kernel():
    mesh = plsc.VectorSubcoreMesh(core_axis_name="c", subcore_axis_name="s",
                                  num_cores=NC, num_subcores=NS)

    @functools.partial(
        pl.kernel,
        out_type=jax.ShapeDtypeStruct((NC, N_SP, W_AGG), jnp.float32),
        mesh=mesh,
        scratch_types=[
            pltpu.VMEM((K // KG, KG * CHUNK), jnp.int32),
            pltpu.VMEM((K // KG, KG * CHUNK), jnp.int32),
            pltpu.VMEM((KG * CHUNK, W_AGG), jnp.float32),
            pltpu.VMEM((ROWS_PER_TILE, W_AGG), jnp.float32),
            pltpu.VMEM_SHARED((N_SP, W_AGG), jnp.float32),
            pltpu.SemaphoreType.DMA,
        ],
        compiler_params=pltpu.CompilerParams(use_tc_tiling_on_sc=False),
    )
    def sc_aggregate(src_hbm, dst_hbm, y2_hbm, zeros_hbm, out_hbm,
                     src_v, dst_v, rows_v, slab_v, agg_sh, sem):
        c = lax.axis_index("c")
        s = lax.axis_index("s")
        wid = s * NC + c
        row0 = s * ROWS_PER_TILE

        pltpu.sync_copy(zeros_hbm.at[pl.ds(row0, ROWS_PER_TILE)], slab_v)
        pltpu.sync_copy(slab_v, agg_sh.at[pl.ds(row0, ROWS_PER_TILE)])
        plsc.subcore_barrier()

        pltpu.sync_copy(src_hbm.at[wid], src_v)
        pltpu.sync_copy(dst_hbm.at[wid], dst_v)

        @pl.loop(0, K // KG)
        def group(g):
            pltpu.async_copy(y2_hbm.at[src_v.at[g]], rows_v, sem).wait()
            pltpu.sync_copy(rows_v, agg_sh.at[dst_v.at[g]], add=True)
        plsc.subcore_barrier()

        pltpu.sync_copy(agg_sh.at[pl.ds(row0, ROWS_PER_TILE)], slab_v)
        pltpu.sync_copy(slab_v, out_hbm.at[c, pl.ds(row0, ROWS_PER_TILE)])

    return sc_aggregate


_SC_AGGREGATE = _make_sc_kernel()


def kernel(x, edge_index, W_l, b_l, W_r):
    ei = edge_index.astype(jnp.int32)
    pad = E_PAD - N_EDGES
    src = jnp.concatenate([ei[0], jnp.zeros((pad,), jnp.int32)])
    dst = jnp.concatenate([ei[1], jnp.full((pad,), N_NODES, jnp.int32)])
    src3 = src.reshape(NW, K // KG, KG * CHUNK)
    dst3 = dst.reshape(NW, K // KG, KG * CHUNK)

    w2 = jnp.concatenate(
        [W_l.T, jnp.zeros((D_FEAT, W_AGG - N_CLASSES), jnp.float32)], axis=1)

    blk = 400
    y2 = pl.pallas_call(
        _proj_body,
        grid=(N_NODES // blk,),
        in_specs=[
            pl.BlockSpec((blk, D_FEAT), lambda i: (i, 0)),
            pl.BlockSpec((D_FEAT, W_AGG), lambda i: (0, 0)),
        ],
        out_specs=pl.BlockSpec((blk, W_AGG), lambda i: (i, 0)),
        out_shape=jax.ShapeDtypeStruct((N_NODES, W_AGG), jnp.float32),
    )(x, w2)

    zeros = jnp.zeros((N_SP, W_AGG), jnp.float32)
    parts = _SC_AGGREGATE(src3, dst3, y2, zeros)

    p0 = parts[0, :N_NODES, :]
    p1 = parts[1, :N_NODES, :]
    out = pl.pallas_call(
        _fin_body,
        grid=(N_NODES // blk,),
        in_specs=[
            pl.BlockSpec((blk, W_AGG), lambda i: (i, 0)),
            pl.BlockSpec((blk, W_AGG), lambda i: (i, 0)),
            pl.BlockSpec((blk, D_FEAT), lambda i: (i, 0)),
            pl.BlockSpec((D_FEAT, N_CLASSES), lambda i: (0, 0)),
            pl.BlockSpec((1, N_CLASSES), lambda i: (0, 0)),
        ],
        out_specs=pl.BlockSpec((blk, N_CLASSES), lambda i: (i, 0)),
        out_shape=jax.ShapeDtypeStruct((N_NODES, N_CLASSES), jnp.float32),
    )(p0, p1, x, W_r.T, b_l.reshape(1, N_CLASSES))
    return out


# speedup vs baseline: 1.8179x; 1.7862x over previous
import functools

import jax
import jax.numpy as jnp
from jax import lax
from jax.experimental import pallas as pl
from jax.experimental.pallas import tpu as pltpu
from jax.experimental.pallas import tpu_sc as plsc

N_NODES = 10000
N_EDGES = 320000
D_FEAT = 128
N_CLASSES = 16

NC = 2
NS = 16
NW = NC * NS
CHUNK = 128
K = 80
KG = 8
E_PAD = NW * K * CHUNK
W_AGG = 2 * N_CLASSES
N_SP = 10112
ROWS_PER_TILE = N_SP // NS


def _proj_body(x_ref, w2_ref, out_ref):
    y = jnp.dot(x_ref[...], w2_ref[...], preferred_element_type=jnp.float32)
    col = lax.broadcasted_iota(jnp.int32, y.shape, 1)
    out_ref[...] = y + jnp.where(col == N_CLASSES, 1.0, 0.0)


def _fin_body(p0_ref, p1_ref, x_ref, wr_ref, b_ref, out_ref):
    ssum = p0_ref[0] + p1_ref[0]
    agg = ssum[:, :N_CLASSES]
    cnt = ssum[:, N_CLASSES:N_CLASSES + 1]
    mean = agg / jnp.maximum(cnt, 1.0)
    z = mean + b_ref[...] + jnp.dot(x_ref[...], wr_ref[...],
                                    preferred_element_type=jnp.float32)
    z = jnp.maximum(z, 0.0)
    m = jnp.max(z, axis=1, keepdims=True)
    lse = m + jnp.log(jnp.sum(jnp.exp(z - m), axis=1, keepdims=True))
    out_ref[...] = z - lse


def _make_sc_kernel():
    mesh = plsc.VectorSubcoreMesh(core_axis_name="c", subcore_axis_name="s",
                                  num_cores=NC, num_subcores=NS)

    @functools.partial(
        pl.kernel,
        out_type=jax.ShapeDtypeStruct((NC, N_SP, W_AGG), jnp.float32),
        mesh=mesh,
        scratch_types=[
            pltpu.VMEM((K // KG, KG * CHUNK), jnp.int32),
            pltpu.VMEM((K // KG, KG * CHUNK), jnp.int32),
            pltpu.VMEM((KG * CHUNK, W_AGG), jnp.float32),
            pltpu.VMEM((ROWS_PER_TILE, W_AGG), jnp.float32),
            pltpu.VMEM_SHARED((N_SP, W_AGG), jnp.float32),
            pltpu.VMEM_SHARED((N_SP, W_AGG), jnp.float32),
            pltpu.SemaphoreType.DMA,
        ],
        compiler_params=pltpu.CompilerParams(use_tc_tiling_on_sc=False),
    )
    def sc_aggregate(src_hbm, dst_hbm, y2_hbm, zeros_hbm, out_hbm,
                     src_v, dst_v, rows_v, slab_v, agg_sh, y_sh, sem):
        c = lax.axis_index("c")
        s = lax.axis_index("s")
        wid = s * NC + c
        row0 = s * ROWS_PER_TILE

        pltpu.sync_copy(zeros_hbm.at[pl.ds(row0, ROWS_PER_TILE)], slab_v)
        pltpu.sync_copy(slab_v, agg_sh.at[pl.ds(row0, ROWS_PER_TILE)])
        pltpu.sync_copy(y2_hbm.at[pl.ds(row0, ROWS_PER_TILE)], slab_v)
        pltpu.sync_copy(slab_v, y_sh.at[pl.ds(row0, ROWS_PER_TILE)])

        pltpu.sync_copy(src_hbm.at[wid], src_v)
        pltpu.sync_copy(dst_hbm.at[wid], dst_v)
        plsc.subcore_barrier()

        @pl.loop(0, K // KG)
        def group(g):
            pltpu.async_copy(y_sh.at[src_v.at[g]], rows_v, sem).wait()
            pltpu.sync_copy(rows_v, agg_sh.at[dst_v.at[g]], add=True)
        plsc.subcore_barrier()

        pltpu.sync_copy(agg_sh.at[pl.ds(row0, ROWS_PER_TILE)], slab_v)
        pltpu.sync_copy(slab_v, out_hbm.at[c, pl.ds(row0, ROWS_PER_TILE)])

    return sc_aggregate


_SC_AGGREGATE = _make_sc_kernel()


def kernel(x, edge_index, W_l, b_l, W_r):
    ei = edge_index.astype(jnp.int32)
    pad = E_PAD - N_EDGES
    src = jnp.concatenate([ei[0], jnp.zeros((pad,), jnp.int32)])
    dst = jnp.concatenate([ei[1], jnp.full((pad,), N_NODES, jnp.int32)])
    src3 = src.reshape(NW, K // KG, KG * CHUNK)
    dst3 = dst.reshape(NW, K // KG, KG * CHUNK)

    w2 = jnp.concatenate(
        [W_l.T, jnp.zeros((D_FEAT, W_AGG - N_CLASSES), jnp.float32)], axis=1)

    blk = 2000
    y2 = pl.pallas_call(
        _proj_body,
        grid=(N_NODES // blk,),
        in_specs=[
            pl.BlockSpec((blk, D_FEAT), lambda i: (i, 0)),
            pl.BlockSpec((D_FEAT, W_AGG), lambda i: (0, 0)),
        ],
        out_specs=pl.BlockSpec((blk, W_AGG), lambda i: (i, 0)),
        out_shape=jax.ShapeDtypeStruct((N_SP, W_AGG), jnp.float32),
    )(x, w2)

    zeros = jnp.zeros((N_SP, W_AGG), jnp.float32)
    parts = _SC_AGGREGATE(src3, dst3, y2, zeros)

    out = pl.pallas_call(
        _fin_body,
        grid=(N_NODES // blk,),
        in_specs=[
            pl.BlockSpec((1, blk, W_AGG), lambda i: (0, i, 0)),
            pl.BlockSpec((1, blk, W_AGG), lambda i: (1, i, 0)),
            pl.BlockSpec((blk, D_FEAT), lambda i: (i, 0)),
            pl.BlockSpec((D_FEAT, N_CLASSES), lambda i: (0, 0)),
            pl.BlockSpec((1, N_CLASSES), lambda i: (0, 0)),
        ],
        out_specs=pl.BlockSpec((blk, N_CLASSES), lambda i: (i, 0)),
        out_shape=jax.ShapeDtypeStruct((N_NODES, N_CLASSES), jnp.float32),
    )(parts, parts, x, W_r.T, b_l.reshape(1, N_CLASSES))
    return out


# speedup vs baseline: 1.8664x; 1.0267x over previous
import functools

import jax
import jax.numpy as jnp
from jax import lax
from jax.experimental import pallas as pl
from jax.experimental.pallas import tpu as pltpu
from jax.experimental.pallas import tpu_sc as plsc

N_NODES = 10000
N_EDGES = 320000
D_FEAT = 128
N_CLASSES = 16

NC = 2
NS = 16
NW = NC * NS
GS = 1000
G = 10
N_SP = 10112
ROWS_PER_TILE = N_SP // NS


def _proj_body(x_ref, w_ref, out_ref):
    out_ref[...] = jnp.dot(x_ref[...], w_ref[...],
                           preferred_element_type=jnp.float32)


def _fin_body(p0_ref, p1_ref, c0_ref, c1_ref, x_ref, wr_ref, b_ref, out_ref):
    agg = p0_ref[0] + p1_ref[0]
    cnt = c0_ref[0][:, :1] + c1_ref[0][:, :1]
    mean = agg / jnp.maximum(cnt, 1.0)
    z = mean + b_ref[...] + jnp.dot(x_ref[...], wr_ref[...],
                                    preferred_element_type=jnp.float32)
    z = jnp.maximum(z, 0.0)
    m = jnp.max(z, axis=1, keepdims=True)
    lse = m + jnp.log(jnp.sum(jnp.exp(z - m), axis=1, keepdims=True))
    out_ref[...] = z - lse


def _make_sc_kernel():
    mesh = plsc.VectorSubcoreMesh(core_axis_name="c", subcore_axis_name="s",
                                  num_cores=NC, num_subcores=NS)

    @functools.partial(
        pl.kernel,
        out_type=(
            jax.ShapeDtypeStruct((NC, N_SP, N_CLASSES), jnp.float32),
            jax.ShapeDtypeStruct((NC, N_SP, 16), jnp.float32),
        ),
        mesh=mesh,
        scratch_types=[
            pltpu.VMEM((G, GS), jnp.int32),
            pltpu.VMEM((G, GS), jnp.int32),
            pltpu.VMEM((GS, N_CLASSES), jnp.float32),
            pltpu.VMEM((GS, 16), jnp.float32),
            pltpu.VMEM((ROWS_PER_TILE, N_CLASSES), jnp.float32),
            pltpu.VMEM((ROWS_PER_TILE, 16), jnp.float32),
            pltpu.VMEM_SHARED((N_SP, N_CLASSES), jnp.float32),
            pltpu.VMEM_SHARED((N_SP, N_CLASSES), jnp.float32),
            pltpu.VMEM_SHARED((N_SP, 16), jnp.float32),
            pltpu.SemaphoreType.DMA,
        ],
        compiler_params=pltpu.CompilerParams(use_tc_tiling_on_sc=False),
    )
    def sc_aggregate(ei_hbm, y_hbm, zf_hbm, zc_hbm, ones_hbm,
                     agg_hbm, cnt_hbm,
                     src_v, dst_v, rows_v, ones_v, fslab_v, cslab_v,
                     y_sh, agg_sh, cnt_sh, sem):
        c = lax.axis_index("c")
        s = lax.axis_index("s")
        wid = s * NC + c
        row0 = s * ROWS_PER_TILE
        e0 = wid * (G * GS)

        pltpu.sync_copy(y_hbm.at[pl.ds(row0, ROWS_PER_TILE)], fslab_v)
        pltpu.sync_copy(fslab_v, y_sh.at[pl.ds(row0, ROWS_PER_TILE)])
        pltpu.sync_copy(zf_hbm.at[pl.ds(row0, ROWS_PER_TILE)], fslab_v)
        pltpu.sync_copy(fslab_v, agg_sh.at[pl.ds(row0, ROWS_PER_TILE)])
        pltpu.sync_copy(zc_hbm.at[pl.ds(row0, ROWS_PER_TILE)], cslab_v)
        pltpu.sync_copy(cslab_v, cnt_sh.at[pl.ds(row0, ROWS_PER_TILE)])
        pltpu.sync_copy(ones_hbm, ones_v)
        for g in range(G):
            pltpu.sync_copy(ei_hbm.at[0, pl.ds(e0 + g * GS, GS)], src_v.at[g])
            pltpu.sync_copy(ei_hbm.at[1, pl.ds(e0 + g * GS, GS)], dst_v.at[g])
        plsc.subcore_barrier()

        @pl.loop(0, G)
        def group(g):
            pltpu.async_copy(y_sh.at[src_v.at[g]], rows_v, sem).wait()
            pltpu.sync_copy(rows_v, agg_sh.at[dst_v.at[g]], add=True)
            pltpu.sync_copy(ones_v, cnt_sh.at[dst_v.at[g]], add=True)
        plsc.subcore_barrier()

        pltpu.sync_copy(agg_sh.at[pl.ds(row0, ROWS_PER_TILE)], fslab_v)
        pltpu.sync_copy(fslab_v, agg_hbm.at[c, pl.ds(row0, ROWS_PER_TILE)])
        pltpu.sync_copy(cnt_sh.at[pl.ds(row0, ROWS_PER_TILE)], cslab_v)
        pltpu.sync_copy(cslab_v, cnt_hbm.at[c, pl.ds(row0, ROWS_PER_TILE)])

    return sc_aggregate


_SC_AGGREGATE = _make_sc_kernel()


def kernel(x, edge_index, W_l, b_l, W_r):
    ei = edge_index.astype(jnp.int32)

    blk = 2000
    y = pl.pallas_call(
        _proj_body,
        grid=(N_NODES // blk,),
        in_specs=[
            pl.BlockSpec((blk, D_FEAT), lambda i: (i, 0)),
            pl.BlockSpec((D_FEAT, N_CLASSES), lambda i: (0, 0)),
        ],
        out_specs=pl.BlockSpec((blk, N_CLASSES), lambda i: (i, 0)),
        out_shape=jax.ShapeDtypeStruct((N_SP, N_CLASSES), jnp.float32),
    )(x, W_l.T)

    zf = jnp.zeros((N_SP, N_CLASSES), jnp.float32)
    zc = jnp.zeros((N_SP, 16), jnp.float32)
    ones = jnp.ones((GS, 16), jnp.float32)
    agg, cnt = _SC_AGGREGATE(ei, y, zf, zc, ones)

    out = pl.pallas_call(
        _fin_body,
        grid=(N_NODES // blk,),
        in_specs=[
            pl.BlockSpec((1, blk, N_CLASSES), lambda i: (0, i, 0)),
            pl.BlockSpec((1, blk, N_CLASSES), lambda i: (1, i, 0)),
            pl.BlockSpec((1, blk, 16), lambda i: (0, i, 0)),
            pl.BlockSpec((1, blk, 16), lambda i: (1, i, 0)),
            pl.BlockSpec((blk, D_FEAT), lambda i: (i, 0)),
            pl.BlockSpec((D_FEAT, N_CLASSES), lambda i: (0, 0)),
            pl.BlockSpec((1, N_CLASSES), lambda i: (0, 0)),
        ],
        out_specs=pl.BlockSpec((blk, N_CLASSES), lambda i: (i, 0)),
        out_shape=jax.ShapeDtypeStruct((N_NODES, N_CLASSES), jnp.float32),
    )(agg, agg, cnt, cnt, x, W_r.T, b_l.reshape(1, N_CLASSES))
    return out


# speedup vs baseline: 2.1951x; 1.1761x over previous
import functools

import jax
import jax.numpy as jnp
from jax import lax
from jax.experimental import pallas as pl
from jax.experimental.pallas import tpu as pltpu
from jax.experimental.pallas import tpu_sc as plsc

N_NODES = 10000
N_EDGES = 320000
D_FEAT = 128
N_CLASSES = 16

NC = 2
NS = 16
NW = NC * NS
GS = 1000
G = 10
N_SP = 10112
ROWS_PER_TILE = N_SP // NS


def _proj_body(x_ref, w2_ref, y_ref, r_ref):
    y2 = jnp.dot(x_ref[...], w2_ref[...], preferred_element_type=jnp.float32)
    y_ref[:, :N_CLASSES] = y2[:, :N_CLASSES]
    r_ref[...] = y2[:, N_CLASSES:]


def _fin_body(p0_ref, p1_ref, r_ref, b_ref, out_ref):
    psum = p0_ref[0] + p1_ref[0]
    agg = psum[:, :N_CLASSES]
    cnt = psum[:, N_CLASSES:N_CLASSES + 1]
    mean = agg / jnp.maximum(cnt, 1.0)
    z = jnp.maximum(mean + b_ref[...] + r_ref[...], 0.0)
    m = jnp.max(z, axis=1, keepdims=True)
    lse = m + jnp.log(jnp.sum(jnp.exp(z - m), axis=1, keepdims=True))
    out_ref[...] = z - lse


def _make_sc_kernel():
    mesh = plsc.VectorSubcoreMesh(core_axis_name="c", subcore_axis_name="s",
                                  num_cores=NC, num_subcores=NS)

    @functools.partial(
        pl.kernel,
        out_type=jax.ShapeDtypeStruct((NC, N_SP, D_FEAT), jnp.float32),
        mesh=mesh,
        scratch_types=[
            pltpu.VMEM((G, GS), jnp.int32),
            pltpu.VMEM((G, GS), jnp.int32),
            pltpu.VMEM((GS, N_CLASSES), jnp.float32),
            pltpu.VMEM((GS, N_CLASSES), jnp.float32),
            pltpu.VMEM((ROWS_PER_TILE, N_CLASSES), jnp.float32),
            pltpu.VMEM((ROWS_PER_TILE, N_CLASSES), jnp.float32),
            pltpu.VMEM_SHARED((N_SP, N_CLASSES), jnp.float32),
            pltpu.VMEM_SHARED((N_SP, N_CLASSES), jnp.float32),
            pltpu.VMEM_SHARED((N_SP, N_CLASSES), jnp.float32),
            pltpu.SemaphoreType.DMA,
        ],
        compiler_params=pltpu.CompilerParams(use_tc_tiling_on_sc=False),
    )
    def sc_aggregate(ei_hbm, y_hbm, zf_hbm, ones_hbm, out_hbm,
                     src_v, dst_v, rows_v, ones_v, fslab_v, cslab_v,
                     y_sh, agg_sh, cnt_sh, sem):
        c = lax.axis_index("c")
        s = lax.axis_index("s")
        wid = s * NC + c
        row0 = s * ROWS_PER_TILE
        e0 = wid * (G * GS)

        pltpu.sync_copy(
            y_hbm.at[pl.ds(row0, ROWS_PER_TILE), pl.ds(0, N_CLASSES)], fslab_v)
        pltpu.sync_copy(fslab_v, y_sh.at[pl.ds(row0, ROWS_PER_TILE)])
        pltpu.sync_copy(zf_hbm.at[pl.ds(row0, ROWS_PER_TILE)], cslab_v)
        pltpu.sync_copy(cslab_v, agg_sh.at[pl.ds(row0, ROWS_PER_TILE)])
        pltpu.sync_copy(cslab_v, cnt_sh.at[pl.ds(row0, ROWS_PER_TILE)])
        pltpu.sync_copy(ones_hbm, ones_v)
        for g in range(G):
            pltpu.sync_copy(ei_hbm.at[0, pl.ds(e0 + g * GS, GS)], src_v.at[g])
            pltpu.sync_copy(ei_hbm.at[1, pl.ds(e0 + g * GS, GS)], dst_v.at[g])
        plsc.subcore_barrier()

        @pl.loop(0, G)
        def group(g):
            pltpu.async_copy(y_sh.at[src_v.at[g]], rows_v, sem).wait()
            pltpu.sync_copy(rows_v, agg_sh.at[dst_v.at[g]], add=True)
            pltpu.sync_copy(ones_v, cnt_sh.at[dst_v.at[g]], add=True)
        plsc.subcore_barrier()

        pltpu.sync_copy(agg_sh.at[pl.ds(row0, ROWS_PER_TILE)], fslab_v)
        pltpu.sync_copy(cnt_sh.at[pl.ds(row0, ROWS_PER_TILE)], cslab_v)
        pltpu.sync_copy(fslab_v, out_hbm.at[c, pl.ds(row0, ROWS_PER_TILE),
                                            pl.ds(0, N_CLASSES)])
        pltpu.sync_copy(cslab_v, out_hbm.at[c, pl.ds(row0, ROWS_PER_TILE),
                                            pl.ds(N_CLASSES, N_CLASSES)])

    return sc_aggregate


_SC_AGGREGATE = _make_sc_kernel()


def kernel(x, edge_index, W_l, b_l, W_r):
    ei = edge_index.astype(jnp.int32)
    w2 = jnp.concatenate([W_l.T, W_r.T], axis=1)

    blk = 2000
    y, r = pl.pallas_call(
        _proj_body,
        grid=(N_NODES // blk,),
        in_specs=[
            pl.BlockSpec((blk, D_FEAT), lambda i: (i, 0)),
            pl.BlockSpec((D_FEAT, 2 * N_CLASSES), lambda i: (0, 0)),
        ],
        out_specs=[
            pl.BlockSpec((blk, D_FEAT), lambda i: (i, 0)),
            pl.BlockSpec((blk, N_CLASSES), lambda i: (i, 0)),
        ],
        out_shape=[
            jax.ShapeDtypeStruct((N_SP, D_FEAT), jnp.float32),
            jax.ShapeDtypeStruct((N_NODES, N_CLASSES), jnp.float32),
        ],
    )(x, w2)

    zf = jnp.zeros((N_SP, N_CLASSES), jnp.float32)
    ones = jnp.ones((GS, N_CLASSES), jnp.float32)
    parts = _SC_AGGREGATE(ei, y, zf, ones)

    out = pl.pallas_call(
        _fin_body,
        grid=(N_NODES // blk,),
        in_specs=[
            pl.BlockSpec((1, blk, D_FEAT), lambda i: (0, i, 0)),
            pl.BlockSpec((1, blk, D_FEAT), lambda i: (1, i, 0)),
            pl.BlockSpec((blk, N_CLASSES), lambda i: (i, 0)),
            pl.BlockSpec((1, N_CLASSES), lambda i: (0, 0)),
        ],
        out_specs=pl.BlockSpec((blk, N_CLASSES), lambda i: (i, 0)),
        out_shape=jax.ShapeDtypeStruct((N_NODES, N_CLASSES), jnp.float32),
    )(parts, parts, r, b_l.reshape(1, N_CLASSES))
    return out


# speedup vs baseline: 2.3342x; 1.0634x over previous
import functools

import jax
import jax.numpy as jnp
from jax import lax
from jax.experimental import pallas as pl
from jax.experimental.pallas import tpu as pltpu
from jax.experimental.pallas import tpu_sc as plsc

N_NODES = 10000
N_EDGES = 320000
D_FEAT = 128
N_CLASSES = 16

NC = 2
NS = 16
NW = NC * NS
GS = 1024
G = 10
E_PAD = 327680
W_CNT = 8
N_SP = 10112
ROWS_PER_TILE = N_SP // NS
BLK = 2000
BLK_F = 2048
EB = E_PAD // (N_NODES // BLK)


def _proj_body(x_ref, ei_ref, w2_ref, y_ref, r_ref, src_ref, dst_ref):
    y2 = jnp.dot(x_ref[...], w2_ref[...], preferred_element_type=jnp.float32)
    y_ref[:, :N_CLASSES] = y2[:, :N_CLASSES]
    r_ref[...] = y2[:, N_CLASSES:]
    i = pl.program_id(0)
    col = lax.broadcasted_iota(jnp.int32, (2, EB), 1)
    valid = col + i * EB < N_EDGES
    row = lax.broadcasted_iota(jnp.int32, (2, EB), 0)
    fill = jnp.where(row == 0, 0, N_NODES)
    sane = jnp.where(valid, ei_ref[...], fill)
    src_ref[...] = sane[0]
    dst_ref[...] = sane[1]


def _fin_body(p0_ref, p1_ref, r_ref, b_ref, out_ref):
    psum = p0_ref[0] + p1_ref[0]
    agg = psum[:, :N_CLASSES]
    cnt = psum[:, N_CLASSES:N_CLASSES + 1]
    mean = agg / jnp.maximum(cnt, 1.0)
    z = jnp.maximum(mean + b_ref[...] + r_ref[...], 0.0)
    m = jnp.max(z, axis=1, keepdims=True)
    lse = m + jnp.log(jnp.sum(jnp.exp(z - m), axis=1, keepdims=True))
    out_ref[...] = (z - lse).T


def _make_sc_kernel():
    mesh = plsc.VectorSubcoreMesh(core_axis_name="c", subcore_axis_name="s",
                                  num_cores=NC, num_subcores=NS)

    @functools.partial(
        pl.kernel,
        out_type=jax.ShapeDtypeStruct((NC, N_SP, D_FEAT), jnp.float32),
        mesh=mesh,
        scratch_types=[
            pltpu.VMEM((G, GS), jnp.int32),
            pltpu.VMEM((G, GS), jnp.int32),
            pltpu.VMEM((2, GS, N_CLASSES), jnp.float32),
            pltpu.VMEM((GS, W_CNT), jnp.float32),
            pltpu.VMEM((ROWS_PER_TILE, N_CLASSES), jnp.float32),
            pltpu.VMEM((ROWS_PER_TILE, W_CNT), jnp.float32),
            pltpu.VMEM_SHARED((N_SP, N_CLASSES), jnp.float32),
            pltpu.VMEM_SHARED((N_SP, N_CLASSES), jnp.float32),
            pltpu.VMEM_SHARED((N_SP, W_CNT), jnp.float32),
            pltpu.SemaphoreType.DMA((2,)),
        ],
        compiler_params=pltpu.CompilerParams(use_tc_tiling_on_sc=False),
    )
    def sc_aggregate(src_hbm, dst_hbm, y_hbm, zf_hbm, ones_hbm, out_hbm,
                     src_v, dst_v, rows_v, ones_v, fslab_v, cslab_v,
                     y_sh, agg_sh, cnt_sh, sems):
        c = lax.axis_index("c")
        s = lax.axis_index("s")
        wid = s * NC + c
        row0 = s * ROWS_PER_TILE
        e0 = wid * (G * GS)

        pltpu.sync_copy(
            y_hbm.at[pl.ds(row0, ROWS_PER_TILE), pl.ds(0, N_CLASSES)], fslab_v)
        pltpu.sync_copy(fslab_v, y_sh.at[pl.ds(row0, ROWS_PER_TILE)])
        pltpu.sync_copy(zf_hbm.at[pl.ds(row0, ROWS_PER_TILE)], fslab_v)
        pltpu.sync_copy(fslab_v, agg_sh.at[pl.ds(row0, ROWS_PER_TILE)])
        pltpu.sync_copy(
            zf_hbm.at[pl.ds(row0, ROWS_PER_TILE), pl.ds(0, W_CNT)], cslab_v)
        pltpu.sync_copy(cslab_v, cnt_sh.at[pl.ds(row0, ROWS_PER_TILE)])
        pltpu.sync_copy(ones_hbm, ones_v)
        for g in range(G):
            pltpu.sync_copy(src_hbm.at[pl.ds(e0 + g * GS, GS)], src_v.at[g])
            pltpu.sync_copy(dst_hbm.at[pl.ds(e0 + g * GS, GS)], dst_v.at[g])
        plsc.subcore_barrier()

        descs = [None] * G
        descs[0] = pltpu.async_copy(y_sh.at[src_v.at[0]], rows_v.at[0],
                                    sems.at[0])
        for g in range(G):
            if g + 1 < G:
                descs[g + 1] = pltpu.async_copy(
                    y_sh.at[src_v.at[g + 1]], rows_v.at[(g + 1) % 2],
                    sems.at[(g + 1) % 2])
            descs[g].wait()
            pltpu.sync_copy(rows_v.at[g % 2], agg_sh.at[dst_v.at[g]], add=True)
            pltpu.sync_copy(ones_v, cnt_sh.at[dst_v.at[g]], add=True)
        plsc.subcore_barrier()

        pltpu.sync_copy(agg_sh.at[pl.ds(row0, ROWS_PER_TILE)], fslab_v)
        pltpu.sync_copy(cnt_sh.at[pl.ds(row0, ROWS_PER_TILE)], cslab_v)
        pltpu.sync_copy(fslab_v, out_hbm.at[c, pl.ds(row0, ROWS_PER_TILE),
                                            pl.ds(0, N_CLASSES)])
        pltpu.sync_copy(cslab_v, out_hbm.at[c, pl.ds(row0, ROWS_PER_TILE),
                                            pl.ds(N_CLASSES, W_CNT)])

    return sc_aggregate


_SC_AGGREGATE = _make_sc_kernel()


def kernel(x, edge_index, W_l, b_l, W_r):
    ei = edge_index.astype(jnp.int32)
    w2 = jnp.concatenate([W_l.T, W_r.T], axis=1)

    y, r, src, dst = pl.pallas_call(
        _proj_body,
        grid=(N_NODES // BLK,),
        in_specs=[
            pl.BlockSpec((BLK, D_FEAT), lambda i: (i, 0)),
            pl.BlockSpec((2, EB), lambda i: (0, i)),
            pl.BlockSpec((D_FEAT, 2 * N_CLASSES), lambda i: (0, 0)),
        ],
        out_specs=[
            pl.BlockSpec((BLK, D_FEAT), lambda i: (i, 0)),
            pl.BlockSpec((BLK, N_CLASSES), lambda i: (i, 0)),
            pl.BlockSpec((EB,), lambda i: (i,)),
            pl.BlockSpec((EB,), lambda i: (i,)),
        ],
        out_shape=[
            jax.ShapeDtypeStruct((N_SP, D_FEAT), jnp.float32),
            jax.ShapeDtypeStruct((N_NODES, N_CLASSES), jnp.float32),
            jax.ShapeDtypeStruct((E_PAD,), jnp.int32),
            jax.ShapeDtypeStruct((E_PAD,), jnp.int32),
        ],
    )(x, ei, w2)

    zf = jnp.zeros((N_SP, N_CLASSES), jnp.float32)
    ones = jnp.ones((GS, W_CNT), jnp.float32)
    parts = _SC_AGGREGATE(src, dst, y, zf, ones)

    out_t = pl.pallas_call(
        _fin_body,
        grid=(5,),
        in_specs=[
            pl.BlockSpec((1, BLK_F, D_FEAT), lambda i: (0, i, 0)),
            pl.BlockSpec((1, BLK_F, D_FEAT), lambda i: (1, i, 0)),
            pl.BlockSpec((BLK_F, N_CLASSES), lambda i: (i, 0)),
            pl.BlockSpec((1, N_CLASSES), lambda i: (0, 0)),
        ],
        out_specs=pl.BlockSpec((N_CLASSES, BLK_F), lambda i: (0, i)),
        out_shape=jax.ShapeDtypeStruct((N_CLASSES, 5 * BLK_F), jnp.float32),
    )(parts, parts, r, b_l.reshape(1, N_CLASSES))
    return out_t[:, :N_NODES].T
